# Initial kernel scaffold; baseline (speedup 1.0000x reference)
import jax
import jax.numpy as jnp
from jax.experimental import pallas as pl


def kernel(inp_mm, x_mask, y_mask):
    raise NotImplementedError("write your pallas kernel here")


# speedup vs baseline: 15.0252x; 15.0252x over previous
import jax
import jax.numpy as jnp
from jax import lax
from jax.experimental import pallas as pl
from jax.experimental.pallas import tpu as pltpu
from jax.experimental.pallas import tpu_sc as plsc

B, XL, YL, C = 8, 512, 512, 8
MIN_VAL = float(-(2**32) + 1)

NW = 32
ROWS_PER_W = (B * XL) // NW
ROW_WORDS = YL * C
CHUNK_ROWS = 8
N_CHUNKS = ROWS_PER_W // CHUNK_ROWS
GROUP = 4
VPR = ROW_WORDS // 16


def _merge3(t1, t2):
    a1, b1, c1 = t1
    a2, b2, c2 = t2
    A = jnp.maximum(a1, a2)
    lo = jnp.minimum(a1, a2)
    hib = jnp.maximum(b1, b2)
    lob = jnp.minimum(b1, b2)
    hic = jnp.maximum(c1, c2)
    Bv = jnp.maximum(lo, hib)
    Cv = jnp.maximum(jnp.maximum(jnp.minimum(hib, lo), lob), hic)
    return A, Bv, Cv


def _insert3(t, x):
    a, b, c = t
    return (jnp.maximum(a, x),
            jnp.maximum(b, jnp.minimum(a, x)),
            jnp.maximum(c, jnp.minimum(b, x)))


def _sc_body(inp, emul_all, eadd_all, xme_all, aux_all, out,
             rowbuf, emul, eadd, xme, auxv, foldsum, foldmax, obuf):
    wid = lax.axis_index("s") * 2 + lax.axis_index("c")
    row0 = wid * ROWS_PER_W
    b = row0 // XL

    zeros = jnp.zeros((16,), jnp.float32)
    minv = jnp.full((16,), MIN_VAL, jnp.float32)
    third = jnp.float32(1.0 / 3.0)

    pltpu.sync_copy(emul_all.at[pl.ds(b * ROW_WORDS, ROW_WORDS)], emul)
    pltpu.sync_copy(eadd_all.at[pl.ds(b * ROW_WORDS, ROW_WORDS)], eadd)
    pltpu.sync_copy(xme_all.at[pl.ds(row0 * 16, ROWS_PER_W * 16)], xme)
    pltpu.sync_copy(aux_all.at[pl.ds(b * 48, 48)], auxv)

    invy = auxv[pl.ds(0, 16)]
    p1 = auxv[pl.ds(16, 16)]
    p3 = auxv[pl.ds(32, 16)]

    foldsum[pl.ds(16, 16)] = zeros
    foldmax[pl.ds(16, 16)] = minv

    def _chunk(ci, _):
        row_start = row0 + ci * CHUNK_ROWS
        pltpu.sync_copy(
            inp.at[pl.ds(row_start * ROW_WORDS, CHUNK_ROWS * ROW_WORDS)],
            rowbuf)

        def _group(g, _):
            base = g * GROUP * ROW_WORDS

            def _acc(i, carry):
                off = i * 16
                em = emul[pl.ds(off, 16)]
                ea = eadd[pl.ds(off, 16)]
                new = []
                for r in range(GROUP):
                    s, a, bb, cc = carry[r]
                    v = rowbuf[pl.ds(base + r * ROW_WORDS + off, 16)]
                    s = s + v * em
                    a, bb, cc = _insert3((a, bb, cc), v + ea)
                    new.append((s, a, bb, cc))
                return tuple(new)

            init = tuple((zeros, minv, minv, minv) for _ in range(GROUP))
            accs = lax.fori_loop(0, VPR, _acc, init)

            for r in range(GROUP):
                s, ta, tb, tc = accs[r]
                r_local = ci * CHUNK_ROWS + g * GROUP + r

                foldsum[pl.ds(0, 16)] = s
                s2 = s + foldsum[pl.ds(8, 16)]
                foldmax[pl.ds(0, 16)] = ta
                ash = foldmax[pl.ds(8, 16)]
                foldmax[pl.ds(0, 16)] = tb
                bsh = foldmax[pl.ds(8, 16)]
                foldmax[pl.ds(0, 16)] = tc
                csh = foldmax[pl.ds(8, 16)]
                A, Bv, Cv = _merge3((ta, tb, tc), (ash, bsh, csh))

                mean = s2 * invy
                top1 = mean + (A - mean) * p1
                top3 = mean + ((A + Bv + Cv) * third - mean) * p3
                xv = xme[pl.ds(r_local * 16, 16)]

                o = r_local * 24
                obuf[pl.ds(o, 16)] = top1 * xv
                obuf[pl.ds(o + 8, 16)] = top3 * xv
                obuf[pl.ds(o + 16, 16)] = mean * xv
            return 0

        lax.fori_loop(0, CHUNK_ROWS // GROUP, _group, 0)
        return 0

    lax.fori_loop(0, N_CHUNKS, _chunk, 0)

    pltpu.sync_copy(obuf.at[pl.ds(0, ROWS_PER_W * 24)],
                    out.at[pl.ds(row0 * 24, ROWS_PER_W * 24)])


@jax.jit
def kernel(inp_mm, x_mask, y_mask):
    inp_flat = inp_mm.reshape(B * XL * YL * C)
    ymf = y_mask.astype(jnp.float32)
    xmf = x_mask.astype(jnp.float32)
    emul_all = jnp.broadcast_to(ymf[:, :, None], (B, YL, C)).reshape(-1)
    eadd_all = ((1.0 - ymf[:, :, None]) *
                jnp.float32(MIN_VAL)).astype(jnp.float32)
    eadd_all = jnp.broadcast_to(eadd_all, (B, YL, C)).reshape(-1)
    xme_all = jnp.broadcast_to(xmf[:, :, None], (B, XL, 16)).reshape(-1)

    ylen = jnp.sum(ymf, axis=1)
    invy = 1.0 / ylen
    p1 = (ylen >= 1.0).astype(jnp.float32)
    p3 = (ylen >= 3.0).astype(jnp.float32)
    aux = jnp.stack([invy, p1, p3], axis=1)
    aux_all = jnp.broadcast_to(aux[:, :, None], (B, 3, 16)).reshape(-1)

    mesh = plsc.VectorSubcoreMesh(core_axis_name="c", subcore_axis_name="s")
    run = pl.kernel(
        _sc_body,
        mesh=mesh,
        out_type=jax.ShapeDtypeStruct((B * XL * 24,), jnp.float32),
        scratch_types=[
            pltpu.VMEM((CHUNK_ROWS * ROW_WORDS,), jnp.float32),
            pltpu.VMEM((ROW_WORDS,), jnp.float32),
            pltpu.VMEM((ROW_WORDS,), jnp.float32),
            pltpu.VMEM((ROWS_PER_W * 16,), jnp.float32),
            pltpu.VMEM((48,), jnp.float32),
            pltpu.VMEM((32,), jnp.float32),
            pltpu.VMEM((32,), jnp.float32),
            pltpu.VMEM((ROWS_PER_W * 24 + 8,), jnp.float32),
        ],
    )
    out = run(inp_flat, emul_all, eadd_all, xme_all, aux_all)
    return out.reshape(B, XL, 3, C)


# speedup vs baseline: 184.6360x; 12.2884x over previous
import jax
import jax.numpy as jnp
from jax import lax
from jax.experimental import pallas as pl
from jax.experimental.pallas import tpu as pltpu
from jax.experimental.pallas import tpu_sc as plsc

B, XL, YL, C = 8, 512, 512, 8
MIN_VAL = float(-(2**32) + 1)

NW = 32
ROW_WORDS = YL * C
CHUNK_ROWS = 8
ITERS = YL // 16

SCR = 64
XS = 4 * SCR
XT = XL - XS
N_CHUNKS = SCR // CHUNK_ROWS
OPAD = SCR + 16
XB = 64


def _merge3(t1, t2):
    a1, b1, c1 = t1
    a2, b2, c2 = t2
    A = jnp.maximum(a1, a2)
    lo = jnp.minimum(a1, a2)
    hib = jnp.maximum(b1, b2)
    lob = jnp.minimum(b1, b2)
    hic = jnp.maximum(c1, c2)
    Bv = jnp.maximum(lo, hib)
    Cv = jnp.maximum(jnp.maximum(jnp.minimum(hib, lo), lob), hic)
    return A, Bv, Cv


def _insert3(t, x):
    a, b, c = t
    return (jnp.maximum(a, x),
            jnp.maximum(b, jnp.minimum(a, x)),
            jnp.maximum(c, jnp.minimum(b, x)))


def _sc_body(inp, ym_lin, xm_lin, out,
             rowbufa, rowbufb, ymraw, xmraw, emulv, eaddv, xmfv,
             fzero, fmin, obuf, sema, semb):
    wid = lax.axis_index("s") * 2 + lax.axis_index("c")
    b = lax.shift_right_logical(wid, 2)
    xq = jnp.bitwise_and(wid, 3)
    x0 = xq * SCR
    row0 = b * XL + x0

    zeros = jnp.zeros((16,), jnp.float32)
    minv = jnp.full((16,), MIN_VAL, jnp.float32)
    third = jnp.float32(1.0 / 3.0)

    mask_handles = [
        pltpu.async_copy(ym_lin.at[pl.ds(t * 1024 + b * 128, 128)],
                         ymraw.at[pl.ds(t * 128, 128)], sema)
        for t in range(4)
    ]
    mask_handles.append(pltpu.async_copy(
        xm_lin.at[pl.ds(lax.shift_right_logical(xq, 1) * 1024 + b * 128
                        + jnp.bitwise_and(xq, 1) * SCR, SCR)],
        xmraw.at[pl.ds(0, SCR)], sema))
    for h in mask_handles:
        h.wait()

    fzero[pl.ds(16, 16)] = zeros
    fmin[pl.ds(16, 16)] = minv

    def _fold_sum16(v):
        for k in (8, 4, 2, 1):
            fzero[pl.ds(0, 16)] = v
            v = v + fzero[pl.ds(k, 16)]
        return v

    def _mask(j, acc):
        m = ymraw[pl.ds(j * 16, 16)].astype(jnp.float32)
        emulv[pl.ds(j * 16, 16)] = m
        eaddv[pl.ds(j * 16, 16)] = (1.0 - m) * MIN_VAL
        return acc + m

    def _xmf(j, _):
        xmfv[pl.ds(j * 16, 16)] = xmraw[pl.ds(j * 16, 16)].astype(
            jnp.float32)
        return 0

    ylen = _fold_sum16(lax.fori_loop(0, ITERS, _mask, zeros))
    lax.fori_loop(0, SCR // 16, _xmf, 0)
    invy = 1.0 / ylen
    p1s = jnp.minimum(ylen, 1.0)
    p3s = jnp.minimum(jnp.maximum(ylen - 2.0, 0.0), 1.0)

    _fold_sum = _fold_sum16

    def _fold_top3(t):
        for k in (8, 4, 2, 1):
            a, bb, cc = t
            fmin[pl.ds(0, 16)] = a
            ash = fmin[pl.ds(k, 16)]
            fmin[pl.ds(0, 16)] = bb
            bsh = fmin[pl.ds(k, 16)]
            fmin[pl.ds(0, 16)] = cc
            csh = fmin[pl.ds(k, 16)]
            t = _merge3(t, (ash, bsh, csh))
        return t

    def _compute_chunk(rowbuf, ci):
        def _row(r, _):
            roff = r * ROW_WORDS
            r_local = ci * CHUNK_ROWS + r

            def _acc(i, carry):
                off = i * 16
                em = emulv[pl.ds(off, 16)]
                ea = eaddv[pl.ds(off, 16)]
                boff = (roff + lax.shift_right_logical(i, 3) * 1024
                        + jnp.bitwise_and(i, 7) * 16)
                new = []
                for c in range(C):
                    s, a, bb, cc = carry[c]
                    v = rowbuf[pl.ds(boff + c * 128, 16)]
                    s = s + v * em
                    a, bb, cc = _insert3((a, bb, cc), v + ea)
                    new.append((s, a, bb, cc))
                return tuple(new)

            init = tuple((zeros, minv, minv, minv) for _ in range(C))
            accs = lax.fori_loop(0, ITERS, _acc, init)

            xv = xmfv[pl.ds(r_local, 16)]
            for c in range(C):
                s, ta, tb, tc = accs[c]
                s = _fold_sum(s)
                A, Bv, Cv = _fold_top3((ta, tb, tc))
                mean = s * invy
                top1 = (mean + (A - mean) * p1s) * xv
                top3 = (mean + ((A + Bv + Cv) * third - mean) * p3s) * xv
                obuf[pl.ds((0 * C + c) * OPAD + r_local, 16)] = top1
                obuf[pl.ds((1 * C + c) * OPAD + r_local, 16)] = top3
                obuf[pl.ds((2 * C + c) * OPAD + r_local, 16)] = mean * xv
            return 0

        lax.fori_loop(0, CHUNK_ROWS, _row, 0)

    CW = CHUNK_ROWS * ROW_WORDS

    def _issue(ci, buf, sem):
        return pltpu.async_copy(
            inp.at[pl.ds((row0 + ci * CHUNK_ROWS) * ROW_WORDS, CW)],
            buf, sem)

    def _wait(buf, sem):
        pltpu.make_async_copy(inp.at[pl.ds(0, CW)], buf, sem).wait()

    _issue(0, rowbufa, sema)

    def _dchunk(d, _):
        ci0 = d * 2
        ci1 = ci0 + 1
        _issue(ci1, rowbufb, semb)
        _wait(rowbufa, sema)
        _compute_chunk(rowbufa, ci0)

        @pl.when(d < N_CHUNKS // 2 - 1)
        def _():
            _issue(ci0 + 2, rowbufa, sema)

        _wait(rowbufb, semb)
        _compute_chunk(rowbufb, ci1)
        return 0

    lax.fori_loop(0, N_CHUNKS // 2, _dchunk, 0)

    xt = lax.shift_right_logical(xq * SCR, 7)
    xl0 = jnp.bitwise_and(xq, 128 // SCR - 1) * SCR
    out_handles = []
    for k in range(3):
        for c in range(C):
            obase = (b * (3 * XS * C) + k * (XS * C) + xt * (128 * C)
                     + c * 128 + xl0)
            out_handles.append(
                pltpu.async_copy(obuf.at[pl.ds((k * C + c) * OPAD, SCR)],
                                 out.at[pl.ds(obase, SCR)], sema))
    for h in out_handles:
        h.wait()


def _tc_body(inp_ref, em_ref, ea_ref, xm_ref, out_ref):
    MIN2 = jnp.float32(-(2.0 ** 40))
    v = inp_ref[0]
    em = em_ref[0]
    ea = ea_ref[0]
    xmv = xm_ref[0]

    s = jnp.sum(v * em, axis=(1, 3))
    vm = v + ea
    m1 = jnp.max(vm, axis=(1, 3))
    e1 = vm == m1[:, None, :, None]
    n1 = jnp.sum(e1.astype(jnp.float32), axis=(1, 3))
    v2 = jnp.where(e1, MIN2, vm)
    m2 = jnp.max(v2, axis=(1, 3))
    e2 = v2 == m2[:, None, :, None]
    n2 = jnp.sum(e2.astype(jnp.float32), axis=(1, 3))
    v3 = jnp.where(e2, MIN2, v2)
    m3 = jnp.max(v3, axis=(1, 3))

    k1 = jnp.minimum(n1, 3.0)
    k2 = jnp.minimum(n2, jnp.maximum(3.0 - n1, 0.0))
    k3 = jnp.maximum(3.0 - n1 - n2, 0.0)
    top3m = (m1 * k1 + m2 * k2 + m3 * k3) * jnp.float32(1.0 / 3.0)

    ylen = jnp.sum(em) * jnp.float32(0.125)
    invy = 1.0 / ylen
    p1 = jnp.minimum(ylen, 1.0)
    p3 = jnp.minimum(jnp.maximum(ylen - 2.0, 0.0), 1.0)
    mean = s * invy
    top1o = (mean + (m1 - mean) * p1) * xmv
    top3o = (mean + (top3m - mean) * p3) * xmv
    out_ref[0] = jnp.stack([top1o, top3o, mean * xmv], axis=1)


@jax.jit
def kernel(inp_mm, x_mask, y_mask):
    inp5 = inp_mm.transpose(0, 1, 3, 2).reshape(B, XL, C, 4, 128)
    inp_flat = inp5.transpose(0, 1, 3, 2, 4).reshape(-1)

    ym_lin = y_mask.reshape(B, 4, 128).transpose(1, 0, 2).reshape(-1)
    xm_lin = x_mask.reshape(B, 4, 128).transpose(1, 0, 2).reshape(-1)

    mesh = plsc.VectorSubcoreMesh(core_axis_name="c", subcore_axis_name="s")
    run = pl.kernel(
        _sc_body,
        mesh=mesh,
        out_type=jax.ShapeDtypeStruct((B * XS * 24,), jnp.float32),
        scratch_types=[
            pltpu.VMEM((CHUNK_ROWS * ROW_WORDS,), jnp.float32),
            pltpu.VMEM((CHUNK_ROWS * ROW_WORDS,), jnp.float32),
            pltpu.VMEM((YL,), jnp.int32),
            pltpu.VMEM((SCR,), jnp.int32),
            pltpu.VMEM((YL,), jnp.float32),
            pltpu.VMEM((YL,), jnp.float32),
            pltpu.VMEM((SCR + 16,), jnp.float32),
            pltpu.VMEM((32,), jnp.float32),
            pltpu.VMEM((32,), jnp.float32),
            pltpu.VMEM((24 * OPAD,), jnp.float32),
            pltpu.SemaphoreType.DMA,
            pltpu.SemaphoreType.DMA,
        ],
    )
    sc_out = run(inp_flat, ym_lin, xm_lin)
    sc_part = (sc_out.reshape(B, 3, XS // 128, C, 128)
               .transpose(0, 2, 4, 1, 3)
               .reshape(B, XS, 3, C))

    ymf = y_mask.astype(jnp.float32)
    xmf = x_mask.astype(jnp.float32)
    inp_tc = inp_flat.reshape(B, XL, 4, C, 128)
    em5 = jnp.broadcast_to(
        ymf.reshape(B, 4, 1, 128), (B, 4, C, 128))
    ea5 = jnp.broadcast_to(
        ((1.0 - ymf) * jnp.float32(MIN_VAL)).reshape(B, 4, 1, 128),
        (B, 4, C, 128))
    xm3 = jnp.broadcast_to(xmf[:, :, None], (B, XL, C))

    tc_part = pl.pallas_call(
        _tc_body,
        grid=(B, XT // XB),
        in_specs=[
            pl.BlockSpec((1, XB, 4, C, 128),
                         lambda b, j: (b, XS // XB + j, 0, 0, 0)),
            pl.BlockSpec((1, 4, C, 128), lambda b, j: (b, 0, 0, 0)),
            pl.BlockSpec((1, 4, C, 128), lambda b, j: (b, 0, 0, 0)),
            pl.BlockSpec((1, XB, C), lambda b, j: (b, XS // XB + j, 0)),
        ],
        out_specs=pl.BlockSpec((1, XB, 3, C), lambda b, j: (b, j, 0, 0)),
        out_shape=jax.ShapeDtypeStruct((B, XT, 3, C), jnp.float32),
    )(inp_tc, em5, ea5, xm3)

    return jnp.concatenate([sc_part, tc_part], axis=1)
